# baseline (speedup 1.0000x reference)
import functools

import jax
import jax.numpy as jnp
from jax import lax
from jax.experimental import pallas as pl
from jax.experimental.pallas import tpu as pltpu
from jax.experimental.pallas import tpu_sc as plsc

_B = 16384
_L = 50
_D = 32
_NC = 2
_NS = 16
_NW = _NC * _NS
_BPW = _B // _NW
_C = 2
_CR = _C * _L
_CP = 104
_NCHUNK = _BPW // _C
_IPW = _NCHUNK * _CP

_mesh = plsc.VectorSubcoreMesh(core_axis_name="c", subcore_axis_name="s")


@functools.partial(
    pl.kernel,
    mesh=_mesh,
    compiler_params=pltpu.CompilerParams(
        needs_layout_passes=False, use_tc_tiling_on_sc=False),
    out_type=jax.ShapeDtypeStruct((_B, _D), jnp.float32),
    scratch_types=[
        pltpu.VMEM((_IPW,), jnp.int32),
        pltpu.VMEM((_CP, _D), jnp.float32),
        pltpu.VMEM((_CP, _D), jnp.float32),
        pltpu.VMEM((_CP, _D), jnp.float32),
        pltpu.VMEM((_CP, _D), jnp.float32),
        pltpu.VMEM((_BPW, _D), jnp.float32),
        pltpu.SemaphoreType.DMA,
        pltpu.SemaphoreType.DMA,
        pltpu.SemaphoreType.DMA,
        pltpu.SemaphoreType.DMA,
    ],
)
def _embed_mean(idx_hbm, table_hbm, out_hbm,
                idx_v, rows0, rows1, rows2, rows3, out_v,
                sem0, sem1, sem2, sem3):
    wid = lax.axis_index("s") * _NC + lax.axis_index("c")

    pltpu.sync_copy(idx_hbm.at[pl.ds(wid * _IPW, _IPW)], idx_v)

    row_b = (rows0, rows1, rows2, rows3)
    sem_b = (sem0, sem1, sem2, sem3)

    def load_chunk(g, buf):
        pltpu.async_copy(
            table_hbm.at[idx_v.at[pl.ds(g * _CP, _CP)]], row_b[buf], sem_b[buf])

    def wait_chunk(g, buf):
        pltpu.make_async_copy(
            table_hbm.at[idx_v.at[pl.ds(g * _CP, _CP)]], row_b[buf],
            sem_b[buf]).wait()

    inv = jnp.float32(1.0 / _L)

    def reduce_chunk(g, buf):
        rows = row_b[buf]
        for i in range(_C):
            a0 = jnp.zeros((16,), jnp.float32)
            a1 = jnp.zeros((16,), jnp.float32)
            for j in range(_L):
                r = i * _L + j
                a0 = a0 + rows[r, pl.ds(0, 16)]
                a1 = a1 + rows[r, pl.ds(16, 16)]
            bag = g * _C + i
            out_v[bag, pl.ds(0, 16)] = a0 * inv
            out_v[bag, pl.ds(16, 16)] = a1 * inv

    _NBUF = 4
    for b in range(_NBUF - 1):
        load_chunk(b, b)

    def quad(p, carry):
        for b in range(_NBUF):
            g = _NBUF * p + b

            @pl.when(g + _NBUF - 1 < _NCHUNK)
            def _():
                load_chunk(g + _NBUF - 1, (b + _NBUF - 1) % _NBUF)

            wait_chunk(g, b)
            reduce_chunk(g, b)
        return carry

    lax.fori_loop(0, _NCHUNK // _NBUF, quad, 0)

    pltpu.sync_copy(out_v, out_hbm.at[pl.ds(wid * _BPW, _BPW)])


def kernel(ngrams, weight):
    ng = ngrams.astype(jnp.int32).reshape(_B // _C, _CR)
    ng = jnp.pad(ng, ((0, 0), (0, _CP - _CR))).reshape(-1)
    return _embed_mean(ng, weight)


# speedup vs baseline: 1.4875x; 1.4875x over previous
import functools

import jax
import jax.numpy as jnp
from jax import lax
from jax.experimental import pallas as pl
from jax.experimental.pallas import tpu as pltpu
from jax.experimental.pallas import tpu_sc as plsc

_B = 16384
_L = 50
_D = 32
_NC = 2
_NS = 16
_NW = _NC * _NS
_BPW = _B // _NW
_C = 4
_CR = _C * _L
_S0 = 96
_S1 = _CR - _S0
_NCHUNK = _BPW // _C
_IPW = _BPW * _L

_mesh = plsc.VectorSubcoreMesh(core_axis_name="c", subcore_axis_name="s")


@functools.partial(
    pl.kernel,
    mesh=_mesh,
    compiler_params=pltpu.CompilerParams(
        needs_layout_passes=False, use_tc_tiling_on_sc=False),
    out_type=jax.ShapeDtypeStruct((_B, _D), jnp.float32),
    scratch_types=[
        pltpu.VMEM((_IPW,), jnp.int32),
        pltpu.VMEM((_CR, _D), jnp.float32),
        pltpu.VMEM((_CR, _D), jnp.float32),
        pltpu.VMEM((_BPW, _D), jnp.float32),
        pltpu.SemaphoreType.DMA,
        pltpu.SemaphoreType.DMA,
    ],
)
def _embed_mean(idx_hbm, table_hbm, out_hbm,
                idx_v, rows0, rows1, out_v, sem0, sem1):
    wid = lax.axis_index("s") * _NC + lax.axis_index("c")

    pltpu.sync_copy(idx_hbm.at[pl.ds(wid * _IPW, _IPW)], idx_v)

    row_b = (rows0, rows1)
    sem_b = (sem0, sem1)

    def load_chunk(g, buf):
        base = g * _CR
        pltpu.async_copy(
            table_hbm.at[idx_v.at[pl.ds(base, _S0)]],
            row_b[buf].at[pl.ds(0, _S0)], sem_b[buf])
        pltpu.async_copy(
            table_hbm.at[idx_v.at[pl.ds(base + _S0, _S1)]],
            row_b[buf].at[pl.ds(_S0, _S1)], sem_b[buf])

    def wait_chunk(g, buf):
        base = g * _CR
        pltpu.make_async_copy(
            table_hbm.at[idx_v.at[pl.ds(base, _S0)]],
            row_b[buf].at[pl.ds(0, _S0)], sem_b[buf]).wait()
        pltpu.make_async_copy(
            table_hbm.at[idx_v.at[pl.ds(base + _S0, _S1)]],
            row_b[buf].at[pl.ds(_S0, _S1)], sem_b[buf]).wait()

    inv = jnp.float32(1.0 / _L)

    def reduce_chunk(g, buf):
        rows = row_b[buf]
        for i in range(_C):
            r0 = i * _L
            a0 = jnp.zeros((16,), jnp.float32)
            a1 = jnp.zeros((16,), jnp.float32)
            b0 = jnp.zeros((16,), jnp.float32)
            b1 = jnp.zeros((16,), jnp.float32)
            for j in range(_L // 2):
                ra = r0 + 2 * j
                rb = ra + 1
                a0 = a0 + rows[ra, pl.ds(0, 16)]
                a1 = a1 + rows[ra, pl.ds(16, 16)]
                b0 = b0 + rows[rb, pl.ds(0, 16)]
                b1 = b1 + rows[rb, pl.ds(16, 16)]
            bag = g * _C + i
            out_v[bag, pl.ds(0, 16)] = (a0 + b0) * inv
            out_v[bag, pl.ds(16, 16)] = (a1 + b1) * inv

    load_chunk(0, 0)

    def pair(p, carry):
        for half in range(2):
            g = 2 * p + half
            cur = half
            nxt = 1 - half

            @pl.when(g + 1 < _NCHUNK)
            def _():
                load_chunk(g + 1, nxt)

            wait_chunk(g, cur)
            reduce_chunk(g, cur)
        return carry

    lax.fori_loop(0, _NCHUNK // 2, pair, 0)

    pltpu.sync_copy(out_v, out_hbm.at[pl.ds(wid * _BPW, _BPW)])


def kernel(ngrams, weight):
    ng = ngrams.astype(jnp.int32).reshape(-1)
    return _embed_mean(ng, weight)


# speedup vs baseline: 1.4885x; 1.0006x over previous
import functools

import jax
import jax.numpy as jnp
from jax import lax
from jax.experimental import pallas as pl
from jax.experimental.pallas import tpu as pltpu
from jax.experimental.pallas import tpu_sc as plsc

_B = 16384
_L = 50
_D = 32
_NC = 2
_NS = 16
_NW = _NC * _NS
_BPW = _B // _NW
_C = 4
_CR = _C * _L
_S0 = 96
_S1 = _CR - _S0
_NCHUNK = _BPW // _C
_IPW = _BPW * _L

_mesh = plsc.VectorSubcoreMesh(core_axis_name="c", subcore_axis_name="s")


@functools.partial(
    pl.kernel,
    mesh=_mesh,
    compiler_params=pltpu.CompilerParams(use_tc_tiling_on_sc=False),
    out_type=jax.ShapeDtypeStruct((_B, _D), jnp.float32),
    scratch_types=[
        pltpu.VMEM((_IPW,), jnp.int32),
        pltpu.VMEM((_CR, _D), jnp.float32),
        pltpu.VMEM((_CR, _D), jnp.float32),
        pltpu.VMEM((_BPW, _D), jnp.float32),
        pltpu.SemaphoreType.DMA,
        pltpu.SemaphoreType.DMA,
    ],
)
def _embed_mean(idx_hbm, table_hbm, out_hbm,
                idx_v, rows0, rows1, out_v, sem0, sem1):
    wid = lax.axis_index("s") * _NC + lax.axis_index("c")

    pltpu.sync_copy(idx_hbm.at[pl.ds(wid * _IPW, _IPW)], idx_v)

    row_b = (rows0, rows1)
    sem_b = (sem0, sem1)

    def load_chunk(g, buf):
        base = g * _CR
        pltpu.async_copy(
            table_hbm.at[idx_v.at[pl.ds(base, _S0)]],
            row_b[buf].at[pl.ds(0, _S0)], sem_b[buf])
        pltpu.async_copy(
            table_hbm.at[idx_v.at[pl.ds(base + _S0, _S1)]],
            row_b[buf].at[pl.ds(_S0, _S1)], sem_b[buf])

    def wait_chunk(g, buf):
        base = g * _CR
        pltpu.make_async_copy(
            table_hbm.at[idx_v.at[pl.ds(base, _S0)]],
            row_b[buf].at[pl.ds(0, _S0)], sem_b[buf]).wait()
        pltpu.make_async_copy(
            table_hbm.at[idx_v.at[pl.ds(base + _S0, _S1)]],
            row_b[buf].at[pl.ds(_S0, _S1)], sem_b[buf]).wait()

    inv = jnp.float32(1.0 / _L)

    def reduce_chunk(g, buf):
        rows = row_b[buf]
        for i in range(_C):
            r0 = i * _L
            a0 = jnp.zeros((16,), jnp.float32)
            a1 = jnp.zeros((16,), jnp.float32)
            b0 = jnp.zeros((16,), jnp.float32)
            b1 = jnp.zeros((16,), jnp.float32)
            for j in range(_L // 2):
                ra = r0 + 2 * j
                rb = ra + 1
                a0 = a0 + rows[ra, pl.ds(0, 16)]
                a1 = a1 + rows[ra, pl.ds(16, 16)]
                b0 = b0 + rows[rb, pl.ds(0, 16)]
                b1 = b1 + rows[rb, pl.ds(16, 16)]
            bag = g * _C + i
            out_v[bag, pl.ds(0, 16)] = (a0 + b0) * inv
            out_v[bag, pl.ds(16, 16)] = (a1 + b1) * inv

    load_chunk(0, 0)

    def pair(p, carry):
        for half in range(2):
            g = 2 * p + half
            cur = half
            nxt = 1 - half

            @pl.when(g + 1 < _NCHUNK)
            def _():
                load_chunk(g + 1, nxt)

            wait_chunk(g, cur)
            reduce_chunk(g, cur)
        return carry

    lax.fori_loop(0, _NCHUNK // 2, pair, 0)

    pltpu.sync_copy(out_v, out_hbm.at[pl.ds(wid * _BPW, _BPW)])


def kernel(ngrams, weight):
    ng = ngrams.astype(jnp.int32).reshape(-1)
    return _embed_mean(ng, weight)
